# baseline (speedup 1.0000x reference)
import functools

import jax
import jax.numpy as jnp
from jax import lax
from jax.experimental import pallas as pl
from jax.experimental.pallas import tpu as pltpu
from jax.experimental.pallas import tpu_sc as plsc

NC, NS = 2, 16
NW = NC * NS

B, L, V, D = 4096, 200, 1000000, 64
F = 4
BB = B // NW
NFIX = BB * F
NBUF = 4
TBUF = 2
MFIX = 32
NGRP = MFIX // 16


def _sc_body(ids_hbm, table_hbm, slots_hbm, cnts_hbm, words_hbm, we_hbm,
             out_hbm, idx_v, rows_v, trows_v, fvecs_v, fwords_v, slots_v,
             cnts_v, gsem, ssem, fgsem):
    c = lax.axis_index("c")
    s = lax.axis_index("s")
    w = s * NC + c

    pltpu.sync_copy(ids_hbm.at[w], idx_v)
    pltpu.sync_copy(words_hbm.at[w], fwords_v)
    pltpu.sync_copy(slots_hbm.at[w], slots_v)
    pltpu.sync_copy(cnts_hbm.at[w], cnts_v)
    for j in range(F):
        pltpu.async_copy(we_hbm.at[fwords_v.at[j]],
                         fvecs_v.at[pl.ds(j * BB, BB)], fgsem)
    for j in range(F):
        pltpu.make_async_copy(we_hbm.at[fwords_v.at[0]],
                              fvecs_v.at[pl.ds(0, BB)], fgsem).wait()

    lane = lax.iota(jnp.int32, 16)
    qvs, dvs = [], []
    for t in range(4):
        dfull = lane + 16 * t
        qvs.append(lax.shift_right_logical(dfull, 3))
        dvs.append(lax.bitwise_and(dfull, 7))

    def fire_gather(l, bslot):
        pltpu.async_copy(table_hbm.at[idx_v.at[l]], rows_v.at[bslot],
                         gsem.at[bslot])

    def wait_gather(bslot):
        pltpu.make_async_copy(table_hbm.at[idx_v.at[0]], rows_v.at[bslot],
                              gsem.at[bslot]).wait()

    def fire_store(l, tslot):
        pltpu.async_copy(trows_v.at[tslot], out_hbm.at[l, :, w],
                         ssem.at[tslot])

    def wait_store(tslot):
        pltpu.make_async_copy(trows_v.at[tslot], out_hbm.at[0, :, w],
                              ssem.at[tslot]).wait()

    for bslot in range(NBUF):
        fire_gather(bslot, bslot)

    @pl.loop(0, L, step=NBUF)
    def _group(g0):
        for bslot in range(NBUF):
            l = g0 + bslot
            tslot = bslot % TBUF
            wait_gather(bslot)

            @pl.when(l >= TBUF)
            def _free():
                wait_store(tslot)

            @pl.loop(0, BB, unroll=4)
            def _col(b_lo):
                bv = jnp.full((16,), b_lo, jnp.int32)
                for t in range(4):
                    vals = rows_v[bslot, b_lo, pl.ds(t * 16, 16)]
                    plsc.store_scatter(trows_v.at[tslot], [qvs[t], dvs[t], bv],
                                       vals)

            cnt = cnts_v[l, pl.ds(0, 16)][0]
            for g in range(NGRP):
                @pl.when(cnt > g * 16)
                def _fix_group():
                    sl16 = slots_v[l, pl.ds(g * 16, 16)]
                    c16 = lax.shift_right_logical(sl16, 2)
                    mask = (lane + g * 16) < jnp.full((16,), cnt, jnp.int32)

                    @pl.loop(0, D)
                    def _fix_d(d):
                        vals = plsc.load_gather(
                            fvecs_v, [sl16, jnp.full((16,), d, jnp.int32)])
                        qv = jnp.full((16,), lax.shift_right_logical(d, 3),
                                      jnp.int32)
                        dv = jnp.full((16,), lax.bitwise_and(d, 7), jnp.int32)
                        plsc.store_scatter(trows_v.at[tslot], [qv, dv, c16],
                                           vals, mask=mask)

            fire_store(l, tslot)

            @pl.when(l + NBUF < L)
            def _refill():
                fire_gather(l + NBUF, bslot)

    for tslot in range(TBUF):
        wait_store(tslot)


@jax.jit
def _embed_with_fixes(idsT3, table, slots3, cnts3, words3, word_embeddings):
    mesh = plsc.VectorSubcoreMesh(
        core_axis_name="c", subcore_axis_name="s",
        num_cores=NC, num_subcores=NS)
    return pl.kernel(
        _sc_body,
        out_type=jax.ShapeDtypeStruct((L, D // 8, NW, 8, 128), jnp.float32),
        mesh=mesh,
        compiler_params=pltpu.CompilerParams(
            use_tc_tiling_on_sc=False, needs_layout_passes=False),
        scratch_types=[
            pltpu.VMEM((L, 128), jnp.int32),
            pltpu.VMEM((NBUF, BB, D), jnp.float32),
            pltpu.VMEM((TBUF, 8, 8, 128), jnp.float32),
            pltpu.VMEM((NFIX, D), jnp.float32),
            pltpu.VMEM((F, 128), jnp.int32),
            pltpu.VMEM((L, MFIX), jnp.int32),
            pltpu.VMEM((L, 16), jnp.int32),
            pltpu.SemaphoreType.DMA((NBUF,)),
            pltpu.SemaphoreType.DMA((TBUF,)),
            pltpu.SemaphoreType.DMA,
        ],
    )(idsT3, table, slots3, cnts3, words3, word_embeddings)


def kernel(input_ids, fix_offsets, fix_words, table, word_embeddings):
    idsT3 = input_ids.T.reshape(L, NW, 128).transpose(1, 0, 2)

    f_ids = jnp.arange(F, dtype=jnp.int32)
    eq = fix_offsets[:, :, None] == fix_offsets[:, None, :]
    last = jnp.max(jnp.where(eq, f_ids[None, None, :], -1), axis=2)
    win_words = jnp.take_along_axis(fix_words, last, axis=1)
    words3 = win_words.reshape(NW, F, 128)

    l_arr = fix_offsets.reshape(NW, NFIX)
    slot_ids = jnp.arange(NFIX, dtype=jnp.int32)[None, :].repeat(NW, axis=0)
    order = jnp.argsort(l_arr, axis=1, stable=True)
    sorted_l = jnp.take_along_axis(l_arr, order, axis=1)
    sorted_slots = jnp.take_along_axis(slot_ids, order, axis=1)
    cnts = (l_arr[:, None, :] == jnp.arange(L, dtype=jnp.int32)[None, :, None]
            ).sum(axis=2).astype(jnp.int32)
    starts = jnp.cumsum(cnts, axis=1) - cnts
    pos = jnp.arange(NFIX, dtype=jnp.int32)[None, :] - jnp.take_along_axis(
        starts, sorted_l, axis=1)

    def _fill(sl, sp, ss):
        return jnp.zeros((L, MFIX), jnp.int32).at[sl, sp].set(ss, mode="drop")

    slots3 = jax.vmap(_fill)(sorted_l, pos, sorted_slots)
    cnts3 = jnp.minimum(cnts, MFIX)[:, :, None].repeat(16, axis=2)

    out = _embed_with_fixes(idsT3, table, slots3, cnts3, words3,
                            word_embeddings)
    return out.transpose(2, 4, 0, 1, 3).reshape(B, L, D)
